# baseline (speedup 1.0000x reference)
import functools

import jax
import jax.numpy as jnp
from jax.experimental import pallas as pl
from jax.experimental.pallas import tpu as pltpu

_LANES = 128
_TINY = 1e-30
_CH = 2048
_NBUF = 3


def _ndpm_body(x_hbm, wm_ref, cb_ref, out_ref, buf_ref, sem, *, K, C, B, D):
    nkc = K * C
    nch = B // _CH

    lane = jax.lax.broadcasted_iota(jnp.int32, (1, _LANES), 1)
    lanef = lane.astype(jnp.float32)
    is_logit = lane < nkc
    is_sig = (lane >= nkc) & (lane < nkc + K)

    cvec = cb_ref[0:1, :]
    bvec = cb_ref[1:2, :]
    csum = jnp.sum(cvec)
    logp = jnp.where(is_sig, jnp.log(jnp.where(is_sig, cvec, 1.0)), 0.0) \
        - jnp.where(is_sig, jnp.log(csum), 0.0)
    wmv = wm_ref[...]
    mu2 = jnp.sum(wmv * wmv, axis=0, keepdims=True)
    bias = bvec + jnp.where(is_sig, logp - 0.5 * mu2, 0.0)
    li = jax.lax.broadcasted_iota(jnp.int32, (_LANES, _LANES), 0)
    lj = jax.lax.broadcasted_iota(jnp.int32, (_LANES, _LANES), 1)
    gi = (li * 205) >> 11
    gj = (lj * 205) >> 11
    A = ((gi == gj) & (li < nkc) & (lj < nkc)).astype(jnp.float32)
    P = ((li >= nkc) & (li < nkc + K) & (lj < nkc) &
         (li - nkc == gj)).astype(jnp.float32)
    S = ((li < nkc) & (lj == li - C * gi)).astype(jnp.float32)

    def _copy(j, slot):
        return pltpu.make_async_copy(
            x_hbm.at[pl.ds(j * _CH, _CH), :], buf_ref.at[slot], sem.at[slot])

    for s in range(_NBUF - 1):
        _copy(s, s).start()

    def _step(j, carry):
        slot = jax.lax.rem(j, _NBUF)
        _copy(j, slot).wait()
        nj = j + _NBUF - 1

        @pl.when(nj < nch)
        def _():
            _copy(nj, jax.lax.rem(nj, _NBUF)).start()

        xb = buf_ref[slot]
        g = jnp.dot(xb, wmv, preferred_element_type=jnp.float32) + bias
        x2 = jnp.sum(xb * xb, axis=1, keepdims=True)

        m1 = jnp.max(jnp.where(is_logit, g, -jnp.inf), axis=1, keepdims=True)
        sm = jnp.where(is_sig, g, -jnp.inf)
        m3 = jnp.max(sm, axis=1, keepdims=True)

        ex = jnp.exp(g - jnp.where(is_logit, m1, m3))
        e1 = jnp.where(is_logit, ex, 0.0)
        q = jnp.where(is_sig, ex, 0.0)

        gsum = jnp.dot(e1, A, preferred_element_type=jnp.float32)
        qb = jnp.dot(q, P, preferred_element_type=jnp.float32)
        pm = jnp.dot(e1 * (qb / jnp.maximum(gsum, _TINY)), S,
                     preferred_element_type=jnp.float32)
        outf = m3 + jnp.log(jnp.maximum(pm, _TINY)) - 0.5 * x2

        hit = jnp.where(sm == m3, lanef, float(_LANES))
        asnf = jnp.min(hit, axis=1, keepdims=True) - float(nkc)

        out16 = jnp.where(lane[:, :16] == C, asnf, outf[:, :16])
        out_ref[pl.ds(j * _CH, _CH), :] = out16
        return carry

    jax.lax.fori_loop(0, nch, _step, 0)


def kernel(x, mus, W, b, counts):
    B, D = x.shape
    K1, _, C = W.shape
    K = K1 - 1
    nkc = K * C

    wl = jnp.transpose(W[1:], (1, 0, 2)).reshape(D, nkc)
    wm = jnp.concatenate(
        [wl, mus[1:].T, jnp.zeros((D, _LANES - nkc - K), jnp.float32)], axis=1)
    cvec = jnp.zeros((_LANES,), jnp.float32).at[nkc:nkc + K].set(counts[1:])
    bvec = jnp.zeros((_LANES,), jnp.float32).at[:nkc].set(b[1:].reshape(-1))
    cb = jnp.zeros((8, _LANES), jnp.float32).at[0].set(cvec).at[1].set(bvec)

    o = pl.pallas_call(
        functools.partial(_ndpm_body, K=K, C=C, B=B, D=D),
        in_specs=[
            pl.BlockSpec(memory_space=pltpu.HBM),
            pl.BlockSpec((D, _LANES), lambda: (0, 0)),
            pl.BlockSpec((8, _LANES), lambda: (0, 0)),
        ],
        out_specs=pl.BlockSpec((B, 16), lambda: (0, 0)),
        out_shape=jax.ShapeDtypeStruct((B, 16), jnp.float32),
        scratch_shapes=[
            pltpu.VMEM((_NBUF, _CH, D), jnp.float32),
            pltpu.SemaphoreType.DMA((_NBUF,)),
        ],
    )(x, wm, cb)
    return o[:, :C], o[:, C].astype(jnp.int32)


# speedup vs baseline: 1.0540x; 1.0540x over previous
import functools

import jax
import jax.numpy as jnp
from jax.experimental import pallas as pl
from jax.experimental.pallas import tpu as pltpu

_LANES = 128
_TINY = 1e-30
_CH = 2048
_NBUF = 4


def _ndpm_body(x_hbm, wm_ref, cb_ref, out_ref, buf_ref, sem,
               obuf_ref, osem, *, K, C, B, D):
    nkc = K * C
    nch = B // _CH

    lane = jax.lax.broadcasted_iota(jnp.int32, (1, _LANES), 1)
    lanef = lane.astype(jnp.float32)
    is_logit = lane < nkc
    is_sig = (lane >= nkc) & (lane < nkc + K)

    cvec = cb_ref[0:1, :]
    bvec = cb_ref[1:2, :]
    csum = jnp.sum(cvec)
    logp = jnp.where(is_sig, jnp.log(jnp.where(is_sig, cvec, 1.0)), 0.0) \
        - jnp.where(is_sig, jnp.log(csum), 0.0)
    wmv = wm_ref[...]
    mu2 = jnp.sum(wmv * wmv, axis=0, keepdims=True)
    bias = bvec + jnp.where(is_sig, logp - 0.5 * mu2, 0.0)
    li = jax.lax.broadcasted_iota(jnp.int32, (_LANES, _LANES), 0)
    lj = jax.lax.broadcasted_iota(jnp.int32, (_LANES, _LANES), 1)
    gi = (li * 205) >> 11
    gj = (lj * 205) >> 11
    A = ((gi == gj) & (li < nkc) & (lj < nkc)).astype(jnp.float32)
    P = ((li >= nkc) & (li < nkc + K) & (lj < nkc) &
         (li - nkc == gj)).astype(jnp.float32)
    S = ((li < nkc) & (lj == li - C * gi)).astype(jnp.float32)

    def _copy(j, slot):
        return pltpu.make_async_copy(
            x_hbm.at[pl.ds(j * _CH, _CH), :], buf_ref.at[slot], sem.at[slot])

    def _ocopy(j, slot):
        return pltpu.make_async_copy(
            obuf_ref.at[slot], out_ref.at[pl.ds(j * _CH, _CH), :],
            osem.at[slot])

    for s in range(_NBUF - 1):
        _copy(s, s).start()

    def _step(j, carry):
        slot = jax.lax.rem(j, _NBUF)
        _copy(j, slot).wait()
        nj = j + _NBUF - 1

        @pl.when(nj < nch)
        def _():
            _copy(nj, jax.lax.rem(nj, _NBUF)).start()

        xb = buf_ref[slot]
        g = jnp.dot(xb, wmv, preferred_element_type=jnp.float32) + bias
        x2 = jnp.sum(xb * xb, axis=1, keepdims=True)

        m1 = jnp.max(jnp.where(is_logit, g, -jnp.inf), axis=1, keepdims=True)
        sm = jnp.where(is_sig, g, -jnp.inf)
        m3 = jnp.max(sm, axis=1, keepdims=True)

        ex = jnp.exp(g - jnp.where(is_logit, m1, m3))
        e1 = jnp.where(is_logit, ex, 0.0)
        q = jnp.where(is_sig, ex, 0.0)

        gsum = jnp.dot(e1, A, preferred_element_type=jnp.float32)
        qb = jnp.dot(q, P, preferred_element_type=jnp.float32)
        pm = jnp.dot(e1 * (qb / jnp.maximum(gsum, _TINY)), S,
                     preferred_element_type=jnp.float32)
        outf = m3 + jnp.log(jnp.maximum(pm, _TINY)) - 0.5 * x2

        hit = jnp.where(sm == m3, lanef, float(_LANES))
        asnf = jnp.min(hit, axis=1, keepdims=True) - float(nkc)

        out16 = jnp.where(lane[:, :16] == C, asnf, outf[:, :16])

        oslot = jax.lax.rem(j, 2)

        @pl.when(j >= 2)
        def _():
            _ocopy(j - 2, oslot).wait()

        obuf_ref[oslot] = out16
        _ocopy(j, oslot).start()
        return carry

    jax.lax.fori_loop(0, nch, _step, 0)
    for jj in (nch - 2, nch - 1):
        _ocopy(jj, jj % 2).wait()


def kernel(x, mus, W, b, counts):
    B, D = x.shape
    K1, _, C = W.shape
    K = K1 - 1
    nkc = K * C

    wl = jnp.transpose(W[1:], (1, 0, 2)).reshape(D, nkc)
    wm = jnp.concatenate(
        [wl, mus[1:].T, jnp.zeros((D, _LANES - nkc - K), jnp.float32)], axis=1)
    cvec = jnp.zeros((_LANES,), jnp.float32).at[nkc:nkc + K].set(counts[1:])
    bvec = jnp.zeros((_LANES,), jnp.float32).at[:nkc].set(b[1:].reshape(-1))
    cb = jnp.zeros((8, _LANES), jnp.float32).at[0].set(cvec).at[1].set(bvec)

    o = pl.pallas_call(
        functools.partial(_ndpm_body, K=K, C=C, B=B, D=D),
        in_specs=[
            pl.BlockSpec(memory_space=pltpu.HBM),
            pl.BlockSpec((D, _LANES), lambda: (0, 0)),
            pl.BlockSpec((8, _LANES), lambda: (0, 0)),
        ],
        out_specs=pl.BlockSpec(memory_space=pltpu.HBM),
        out_shape=jax.ShapeDtypeStruct((B, 16), jnp.float32),
        scratch_shapes=[
            pltpu.VMEM((_NBUF, _CH, D), jnp.float32),
            pltpu.SemaphoreType.DMA((_NBUF,)),
            pltpu.VMEM((2, _CH, 16), jnp.float32),
            pltpu.SemaphoreType.DMA((2,)),
        ],
    )(x, wm, cb)
    return o[:, :C], o[:, C].astype(jnp.int32)
